# baseline (speedup 1.0000x reference)
import functools

import jax
import jax.numpy as jnp
from jax import lax
from jax.experimental import pallas as pl
from jax.experimental.pallas import tpu as pltpu
from jax.experimental.pallas import tpu_sc as plsc

NC = 2
NS = 16
NW = NC * NS
CW = 128
G = 64

_MESH = plsc.VectorSubcoreMesh(core_axis_name="c", subcore_axis_name="s")


def _deg_body(n0, n1, degr, dst_hbm, ones_hbm, zeros_hbm, out_hbm,
              dst_v, ones_v, zbuf_v, deg_s):
    c = lax.axis_index("c")
    s = lax.axis_index("s")
    dpt = degr // NS
    pltpu.sync_copy(dst_hbm.at[c, s], dst_v)
    pltpu.sync_copy(ones_hbm, ones_v)
    pltpu.sync_copy(zeros_hbm, zbuf_v)
    pltpu.sync_copy(zbuf_v, deg_s.at[pl.ds(s * dpt, dpt)])
    plsc.subcore_barrier()

    def body(j, carry):
        pltpu.sync_copy(ones_v, deg_s.at[dst_v.at[j]], add=True)
        return carry

    lax.fori_loop(0, jnp.where(c == 0, n0, n1), body, 0)
    plsc.subcore_barrier()
    pltpu.sync_copy(deg_s.at[pl.ds(s * dpt, dpt)], zbuf_v)
    pltpu.sync_copy(zbuf_v, out_hbm.at[pl.ds(c * degr + s * dpt, dpt)])


def _agg_body(n0, n1, accr, hs_hbm, src_hbm, dst_hbm, zeros_hbm, out_hbm,
              srcidx_v, dstidx_v, rows_v, zbuf_v, acc_s,
              sem_ia, sem_ib, sem_ra, sem_rb):
    c = lax.axis_index("c")
    s = lax.axis_index("s")
    rpt = accr // NS
    rcw = rpt // CW
    zr = zbuf_v.shape[0]

    isems = (sem_ia, sem_ib)
    rsems = (sem_ra, sem_rb)

    def idxfire(j, b):
        pltpu.async_copy(src_hbm.at[c, s, j], srcidx_v.at[b], isems[b])
        pltpu.async_copy(dst_hbm.at[c, s, j], dstidx_v.at[b], isems[b])

    def idxwait(b):
        pltpu.make_async_copy(src_hbm.at[0, 0, 0], srcidx_v.at[b],
                              isems[b]).wait()
        pltpu.make_async_copy(dst_hbm.at[0, 0, 0], dstidx_v.at[b],
                              isems[b]).wait()

    def rowfire(b):
        pltpu.async_copy(hs_hbm.at[srcidx_v.at[b]], rows_v.at[b], rsems[b])

    def rowwait(b):
        pltpu.make_async_copy(hs_hbm.at[pl.ds(0, CW)], rows_v.at[b],
                              rsems[b]).wait()

    def scatter(b):
        pltpu.sync_copy(rows_v.at[b], acc_s.at[dstidx_v.at[b]], add=True)

    def pipeline(nchunks):
        def step(j, b):
            @pl.when(j + 1 < nchunks)
            def _():
                idxwait(1 - b)
                rowfire(1 - b)

            rowwait(b)
            scatter(b)

            @pl.when(j + 2 < nchunks)
            def _():
                idxfire(j + 2, b)

        def body(p, carry):
            step(2 * p, 0)
            step(2 * p + 1, 1)
            return carry

        lax.fori_loop(0, nchunks // 2, body, 0)

    idxfire(0, 0)
    idxwait(0)
    rowfire(0)
    idxfire(1, 1)
    pltpu.sync_copy(zeros_hbm, zbuf_v)
    for k in range(rpt // zr):
        pltpu.sync_copy(zbuf_v, acc_s.at[pl.ds(s * rpt + k * zr, zr)])
    plsc.subcore_barrier()

    @pl.when(c == 0)
    def _():
        pipeline(n0)

    @pl.when(c == 1)
    def _():
        pipeline(n1)

    plsc.subcore_barrier()
    for k in range(rcw):
        b = k % 2
        if k >= 2:
            pltpu.make_async_copy(rows_v.at[b], out_hbm.at[c, pl.ds(0, CW)],
                                  isems[b]).wait()
        pltpu.sync_copy(acc_s.at[pl.ds(s * rpt + k * CW, CW)], rows_v.at[b])
        pltpu.async_copy(rows_v.at[b],
                         out_hbm.at[c, pl.ds(s * rpt + k * CW, CW)], isems[b])
    for k in range(max(rcw - 2, 0), rcw):
        b = k % 2
        pltpu.make_async_copy(rows_v.at[b], out_hbm.at[c, pl.ds(0, CW)],
                              isems[b]).wait()


def _dis_tc(degT_ref, out_ref):
    d = jnp.sum(degT_ref[...], axis=1, keepdims=True) + 1.0
    dis = lax.rsqrt(d)
    out_ref[...] = jnp.broadcast_to(dis, out_ref.shape)


def _mm_scale_tc(x_ref, w_ref, dis_ref, out_ref):
    h = jnp.dot(x_ref[...], w_ref[...], preferred_element_type=jnp.float32)
    out_ref[...] = h * dis_ref[...]


def _mid_tc(a0_ref, a1_ref, hs_ref, dis_ref, b_ref, w_ref, out_ref):
    dis = dis_ref[...]
    t = (a0_ref[0] + a1_ref[0] + hs_ref[...]) * dis + b_ref[...]
    t = jnp.where(t >= 0, t, 0.2 * t)
    out_ref[...] = jnp.dot(t, w_ref[...],
                           preferred_element_type=jnp.float32) * dis


def _final_tc(a0_ref, a1_ref, hs_ref, dis_ref, b_ref, batch_ref, wc_ref,
              bc_ref, out_ref, sums, cnts):
    i = pl.program_id(0)
    n = pl.num_programs(0)
    dis = dis_ref[...]
    t = (a0_ref[0] + a1_ref[0] + hs_ref[...]) * dis + b_ref[...]
    t = jnp.where(t >= 0, t, 0.2 * t)
    rows = t.shape[0]
    oh = (lax.broadcasted_iota(jnp.int32, (G, rows), 0)
          == batch_ref[0]).astype(jnp.float32)

    @pl.when(i == 0)
    def _():
        sums[...] = jnp.zeros_like(sums)
        cnts[...] = jnp.zeros_like(cnts)

    sums[...] += jnp.dot(oh, t, preferred_element_type=jnp.float32)
    cnts[...] += jnp.broadcast_to(
        jnp.sum(oh, axis=1, keepdims=True), cnts.shape)

    @pl.when(i == n - 1)
    def _():
        pooled = sums[...] / jnp.maximum(cnts[...], 1.0)
        out_ref[...] = jnp.dot(pooled, wc_ref[...],
                               preferred_element_type=jnp.float32) + bc_ref[...]


def _ceil_to(a, m):
    return -(-a // m) * m


def kernel(x, edge_index, batch, W1, b1, W2, b2, Wc, bc):
    N, D = x.shape
    E = edge_index.shape[1]
    H = W1.shape[1]
    C = Wc.shape[1]
    f32 = jnp.float32

    accr = _ceil_to(N + 1, NS * CW)

    tch = -(-E // (NS * CW))
    n0 = int(round(0.835 * tch))
    n0 += n0 % 2
    n1 = tch - n0
    n1 += n1 % 2
    cap = NS * CW * (n0 + n1)

    src = edge_index[0]
    dst = edge_index[1]
    if cap > E:
        src = jnp.concatenate([src, jnp.zeros((cap - E,), jnp.int32)])
        dst = jnp.concatenate([dst, jnp.full((cap - E,), N, jnp.int32)])

    def _core_layout(flat, fill):
        p0 = flat[: NS * n0 * CW].reshape(1, NS, n0, CW)
        p1 = flat[NS * n0 * CW:].reshape(1, NS, n1, CW)
        pad = jnp.full((1, NS, n0 - n1, CW), fill, jnp.int32)
        return jnp.concatenate([p0, jnp.concatenate([p1, pad], axis=2)])

    src4 = _core_layout(src, 0)
    dst4 = _core_layout(dst, N)

    zeros1 = jnp.zeros((accr // NS,), f32)
    zeros2 = jnp.zeros((64, H), f32)
    ones1 = jnp.ones((CW,), f32)

    deg_kernel = pl.kernel(
        functools.partial(_deg_body, n0, n1, accr),
        out_type=jax.ShapeDtypeStruct((NC * accr,), f32),
        mesh=_MESH,
        scratch_types=[
            pltpu.VMEM((n0, CW), jnp.int32),
            pltpu.VMEM((CW,), f32),
            pltpu.VMEM((accr // NS,), f32),
            pltpu.VMEM_SHARED((accr,), f32),
        ],
    )
    degp = deg_kernel(dst4, ones1, zeros1)

    agg_call = pl.kernel(
        functools.partial(_agg_body, n0, n1, accr),
        out_type=jax.ShapeDtypeStruct((NC, accr, H), f32),
        mesh=_MESH,
        scratch_types=[
            pltpu.VMEM((2, CW), jnp.int32),
            pltpu.VMEM((2, CW), jnp.int32),
            pltpu.VMEM((2, CW, H), f32),
            pltpu.VMEM((64, H), f32),
            pltpu.VMEM_SHARED((accr, H), f32),
            pltpu.SemaphoreType.DMA,
            pltpu.SemaphoreType.DMA,
            pltpu.SemaphoreType.DMA,
            pltpu.SemaphoreType.DMA,
        ],
    )

    degT = jnp.transpose(degp.reshape(NC, accr))
    bn = accr // 8
    dis_b = pl.pallas_call(
        _dis_tc,
        grid=(8,),
        in_specs=[pl.BlockSpec((bn, NC), lambda i: (i, 0))],
        out_specs=pl.BlockSpec((bn, H), lambda i: (i, 0)),
        out_shape=jax.ShapeDtypeStruct((accr, H), f32),
    )(degT)

    BR = 1000
    grid_n = N // BR

    hs1 = pl.pallas_call(
        _mm_scale_tc,
        grid=(grid_n,),
        in_specs=[
            pl.BlockSpec((BR, D), lambda i: (i, 0)),
            pl.BlockSpec((D, H), lambda i: (0, 0)),
            pl.BlockSpec((BR, H), lambda i: (i, 0)),
        ],
        out_specs=pl.BlockSpec((BR, H), lambda i: (i, 0)),
        out_shape=jax.ShapeDtypeStruct((N, H), f32),
    )(x, W1, dis_b)

    acc1 = agg_call(hs1, src4, dst4, zeros2)

    hs2 = pl.pallas_call(
        _mid_tc,
        grid=(grid_n,),
        in_specs=[
            pl.BlockSpec((1, BR, H), lambda i: (0, i, 0)),
            pl.BlockSpec((1, BR, H), lambda i: (1, i, 0)),
            pl.BlockSpec((BR, H), lambda i: (i, 0)),
            pl.BlockSpec((BR, H), lambda i: (i, 0)),
            pl.BlockSpec((1, H), lambda i: (0, 0)),
            pl.BlockSpec((H, H), lambda i: (0, 0)),
        ],
        out_specs=pl.BlockSpec((BR, H), lambda i: (i, 0)),
        out_shape=jax.ShapeDtypeStruct((N, H), f32),
    )(acc1, acc1, hs1, dis_b, b1.reshape(1, H), W2)

    acc2 = agg_call(hs2, src4, dst4, zeros2)

    out = pl.pallas_call(
        _final_tc,
        grid=(grid_n,),
        in_specs=[
            pl.BlockSpec((1, BR, H), lambda i: (0, i, 0)),
            pl.BlockSpec((1, BR, H), lambda i: (1, i, 0)),
            pl.BlockSpec((BR, H), lambda i: (i, 0)),
            pl.BlockSpec((BR, H), lambda i: (i, 0)),
            pl.BlockSpec((1, H), lambda i: (0, 0)),
            pl.BlockSpec((1, 1, BR), lambda i: (i, 0, 0)),
            pl.BlockSpec((H, C), lambda i: (0, 0)),
            pl.BlockSpec((1, C), lambda i: (0, 0)),
        ],
        out_specs=pl.BlockSpec((G, C), lambda i: (0, 0)),
        out_shape=jax.ShapeDtypeStruct((G, C), f32),
        scratch_shapes=[
            pltpu.VMEM((G, H), f32),
            pltpu.VMEM((G, H), f32),
        ],
    )(acc2, acc2, hs2, dis_b, b2.reshape(1, H), batch.reshape(grid_n, 1, BR),
      Wc, bc.reshape(1, C))
    return out


# speedup vs baseline: 1.2324x; 1.2324x over previous
import functools

import jax
import jax.numpy as jnp
from jax import lax
from jax.experimental import pallas as pl
from jax.experimental.pallas import tpu as pltpu
from jax.experimental.pallas import tpu_sc as plsc

NC = 2
NS = 16
NW = NC * NS
CW = 128
G = 64

_MESH = plsc.VectorSubcoreMesh(core_axis_name="c", subcore_axis_name="s")


def _deg_body(n0c, n1c, tc, e0, e1f, degr, dst_hbm, dstt_hbm, ones_hbm,
              zeros_hbm, out_hbm, dst_v, ones_v, zbuf_v, deg_s, sem):
    c = lax.axis_index("c")
    s = lax.axis_index("s")
    dpt = degr // NS
    base = pl.multiple_of(jnp.where(c == 0, s * e0, NS * e0 + s * e1f), 8)
    n = jnp.where(c == 0, n0c, n1c)

    def fill(j, carry):
        pltpu.async_copy(dst_hbm.at[pl.ds(base + j * CW, CW)], dst_v.at[j],
                         sem)
        return carry

    lax.fori_loop(0, n, fill, 0)
    pltpu.sync_copy(ones_hbm, ones_v)
    pltpu.sync_copy(zeros_hbm, zbuf_v)
    pltpu.sync_copy(zbuf_v, deg_s.at[pl.ds(s * dpt, dpt)])
    plsc.subcore_barrier()

    def drain(j, carry):
        pltpu.make_async_copy(dst_hbm.at[pl.ds(0, CW)], dst_v.at[0],
                              sem).wait()
        return carry

    lax.fori_loop(0, n, drain, 0)

    def body(j, carry):
        pltpu.sync_copy(ones_v, deg_s.at[dst_v.at[j]], add=True)
        return carry

    lax.fori_loop(0, n, body, 0)

    @pl.when(jnp.logical_and(c == 1, s < tc))
    def _():
        pltpu.sync_copy(dstt_hbm.at[s], dst_v.at[0])
        pltpu.sync_copy(ones_v, deg_s.at[dst_v.at[0]], add=True)

    plsc.subcore_barrier()
    pltpu.sync_copy(deg_s.at[pl.ds(s * dpt, dpt)], zbuf_v)
    pltpu.sync_copy(zbuf_v, out_hbm.at[pl.ds(c * degr + s * dpt, dpt)])


def _agg_body(n0c, n1c, tc, e0, e1f, accr, hs_hbm, src_hbm, dst_hbm,
              srct_hbm, dstt_hbm, zeros_hbm, out_hbm,
              srcidx_v, dstidx_v, rows_v, zbuf_v, acc_s,
              sem_ia, sem_ib, sem_ra, sem_rb):
    c = lax.axis_index("c")
    s = lax.axis_index("s")
    rpt = accr // NS
    rcw = rpt // CW
    zr = zbuf_v.shape[0]
    base = pl.multiple_of(jnp.where(c == 0, s * e0, NS * e0 + s * e1f), 8)

    isems = (sem_ia, sem_ib)
    rsems = (sem_ra, sem_rb)

    def idxfire(j, b):
        pltpu.async_copy(src_hbm.at[pl.ds(base + j * CW, CW)],
                         srcidx_v.at[b], isems[b])
        pltpu.async_copy(dst_hbm.at[pl.ds(base + j * CW, CW)],
                         dstidx_v.at[b], isems[b])

    def idxwait(b):
        pltpu.make_async_copy(src_hbm.at[pl.ds(0, CW)], srcidx_v.at[b],
                              isems[b]).wait()
        pltpu.make_async_copy(dst_hbm.at[pl.ds(0, CW)], dstidx_v.at[b],
                              isems[b]).wait()

    def rowfire(b):
        pltpu.async_copy(hs_hbm.at[srcidx_v.at[b]], rows_v.at[b], rsems[b])

    def rowwait(b):
        pltpu.make_async_copy(hs_hbm.at[pl.ds(0, CW)], rows_v.at[b],
                              rsems[b]).wait()

    def scatter(b):
        pltpu.sync_copy(rows_v.at[b], acc_s.at[dstidx_v.at[b]], add=True)

    def pipeline(nchunks):
        def step(j, b):
            @pl.when(j + 1 < nchunks)
            def _():
                idxwait(1 - b)
                rowfire(1 - b)

            rowwait(b)
            scatter(b)

            @pl.when(j + 2 < nchunks)
            def _():
                idxfire(j + 2, b)

        def body(p, carry):
            step(2 * p, 0)
            step(2 * p + 1, 1)
            return carry

        lax.fori_loop(0, nchunks // 2, body, 0)

    idxfire(0, 0)
    idxwait(0)
    rowfire(0)
    idxfire(1, 1)
    pltpu.sync_copy(zeros_hbm, zbuf_v)
    for k in range(rpt // zr):
        pltpu.sync_copy(zbuf_v, acc_s.at[pl.ds(s * rpt + k * zr, zr)])
    plsc.subcore_barrier()

    @pl.when(c == 0)
    def _():
        pipeline(n0c)

    @pl.when(c == 1)
    def _():
        pipeline(n1c)

    @pl.when(jnp.logical_and(c == 1, s < tc))
    def _():
        pltpu.sync_copy(srct_hbm.at[s], srcidx_v.at[0])
        pltpu.sync_copy(dstt_hbm.at[s], dstidx_v.at[0])
        pltpu.sync_copy(hs_hbm.at[srcidx_v.at[0]], rows_v.at[0])
        pltpu.sync_copy(rows_v.at[0], acc_s.at[dstidx_v.at[0]], add=True)

    plsc.subcore_barrier()
    for k in range(rcw):
        b = k % 2
        if k >= 2:
            pltpu.make_async_copy(rows_v.at[b], out_hbm.at[c, pl.ds(0, CW)],
                                  isems[b]).wait()
        pltpu.sync_copy(acc_s.at[pl.ds(s * rpt + k * CW, CW)], rows_v.at[b])
        pltpu.async_copy(rows_v.at[b],
                         out_hbm.at[c, pl.ds(s * rpt + k * CW, CW)], isems[b])
    for k in range(max(rcw - 2, 0), rcw):
        b = k % 2
        pltpu.make_async_copy(rows_v.at[b], out_hbm.at[c, pl.ds(0, CW)],
                              isems[b]).wait()


def _dis_tc(degT_ref, out_ref):
    d = jnp.sum(degT_ref[...], axis=1, keepdims=True) + 1.0
    dis = lax.rsqrt(d)
    out_ref[...] = jnp.broadcast_to(dis, out_ref.shape)


def _mm_scale_tc(x_ref, w_ref, dis_ref, out_ref):
    h = jnp.dot(x_ref[...], w_ref[...], preferred_element_type=jnp.float32)
    out_ref[...] = h * dis_ref[...]


def _mid_tc(a0_ref, a1_ref, hs_ref, dis_ref, b_ref, w_ref, out_ref):
    dis = dis_ref[...]
    t = (a0_ref[0] + a1_ref[0] + hs_ref[...]) * dis + b_ref[...]
    t = jnp.where(t >= 0, t, 0.2 * t)
    out_ref[...] = jnp.dot(t, w_ref[...],
                           preferred_element_type=jnp.float32) * dis


def _final_tc(a0_ref, a1_ref, hs_ref, dis_ref, b_ref, batch_ref, wc_ref,
              bc_ref, out_ref, sums, cnts):
    i = pl.program_id(0)
    n = pl.num_programs(0)
    dis = dis_ref[...]
    t = (a0_ref[0] + a1_ref[0] + hs_ref[...]) * dis + b_ref[...]
    t = jnp.where(t >= 0, t, 0.2 * t)
    rows = t.shape[0]
    oh = (lax.broadcasted_iota(jnp.int32, (G, rows), 0)
          == batch_ref[0]).astype(jnp.float32)

    @pl.when(i == 0)
    def _():
        sums[...] = jnp.zeros_like(sums)
        cnts[...] = jnp.zeros_like(cnts)

    sums[...] += jnp.dot(oh, t, preferred_element_type=jnp.float32)
    cnts[...] += jnp.broadcast_to(
        jnp.sum(oh, axis=1, keepdims=True), cnts.shape)

    @pl.when(i == n - 1)
    def _():
        pooled = sums[...] / jnp.maximum(cnts[...], 1.0)
        out_ref[...] = jnp.dot(pooled, wc_ref[...],
                               preferred_element_type=jnp.float32) + bc_ref[...]


def _ceil_to(a, m):
    return -(-a // m) * m


def kernel(x, edge_index, batch, W1, b1, W2, b2, Wc, bc):
    N, D = x.shape
    E = edge_index.shape[1]
    H = W1.shape[1]
    C = Wc.shape[1]
    f32 = jnp.float32

    accr = _ceil_to(N + 1, NS * CW)

    e_pt = E // NS
    e0 = (int(round(0.82 * e_pt)) // (2 * CW)) * (2 * CW)
    n0c = e0 // CW
    e1f = ((e_pt - e0) // (2 * CW)) * (2 * CW)
    n1c = e1f // CW
    tail = E - NS * (e0 + e1f)
    tc = tail // CW

    src_flat = edge_index[0]
    dst_flat = edge_index[1]
    tb = NS * (e0 + e1f)
    src_t = src_flat[tb:].reshape(tc, CW)
    dst_t = dst_flat[tb:].reshape(tc, CW)

    zeros1 = jnp.zeros((accr // NS,), f32)
    zeros2 = jnp.zeros((64, H), f32)
    ones1 = jnp.ones((CW,), f32)

    deg_kernel = pl.kernel(
        functools.partial(_deg_body, n0c, n1c, tc, e0, e1f, accr),
        out_type=jax.ShapeDtypeStruct((NC * accr,), f32),
        mesh=_MESH,
        scratch_types=[
            pltpu.VMEM((n0c, CW), jnp.int32),
            pltpu.VMEM((CW,), f32),
            pltpu.VMEM((accr // NS,), f32),
            pltpu.VMEM_SHARED((accr,), f32),
            pltpu.SemaphoreType.DMA,
        ],
    )
    degp = deg_kernel(dst_flat, dst_t, ones1, zeros1)

    agg_call = pl.kernel(
        functools.partial(_agg_body, n0c, n1c, tc, e0, e1f, accr),
        out_type=jax.ShapeDtypeStruct((NC, accr, H), f32),
        mesh=_MESH,
        scratch_types=[
            pltpu.VMEM((2, CW), jnp.int32),
            pltpu.VMEM((2, CW), jnp.int32),
            pltpu.VMEM((2, CW, H), f32),
            pltpu.VMEM((64, H), f32),
            pltpu.VMEM_SHARED((accr, H), f32),
            pltpu.SemaphoreType.DMA,
            pltpu.SemaphoreType.DMA,
            pltpu.SemaphoreType.DMA,
            pltpu.SemaphoreType.DMA,
        ],
    )

    degT = jnp.transpose(degp.reshape(NC, accr))
    bn = accr // 8
    dis_b = pl.pallas_call(
        _dis_tc,
        grid=(8,),
        in_specs=[pl.BlockSpec((bn, NC), lambda i: (i, 0))],
        out_specs=pl.BlockSpec((bn, H), lambda i: (i, 0)),
        out_shape=jax.ShapeDtypeStruct((accr, H), f32),
    )(degT)

    BR = 1000
    grid_n = N // BR

    hs1 = pl.pallas_call(
        _mm_scale_tc,
        grid=(grid_n,),
        in_specs=[
            pl.BlockSpec((BR, D), lambda i: (i, 0)),
            pl.BlockSpec((D, H), lambda i: (0, 0)),
            pl.BlockSpec((BR, H), lambda i: (i, 0)),
        ],
        out_specs=pl.BlockSpec((BR, H), lambda i: (i, 0)),
        out_shape=jax.ShapeDtypeStruct((N, H), f32),
    )(x, W1, dis_b)

    acc1 = agg_call(hs1, src_flat, dst_flat, src_t, dst_t, zeros2)

    hs2 = pl.pallas_call(
        _mid_tc,
        grid=(grid_n,),
        in_specs=[
            pl.BlockSpec((1, BR, H), lambda i: (0, i, 0)),
            pl.BlockSpec((1, BR, H), lambda i: (1, i, 0)),
            pl.BlockSpec((BR, H), lambda i: (i, 0)),
            pl.BlockSpec((BR, H), lambda i: (i, 0)),
            pl.BlockSpec((1, H), lambda i: (0, 0)),
            pl.BlockSpec((H, H), lambda i: (0, 0)),
        ],
        out_specs=pl.BlockSpec((BR, H), lambda i: (i, 0)),
        out_shape=jax.ShapeDtypeStruct((N, H), f32),
    )(acc1, acc1, hs1, dis_b, b1.reshape(1, H), W2)

    acc2 = agg_call(hs2, src_flat, dst_flat, src_t, dst_t, zeros2)

    out = pl.pallas_call(
        _final_tc,
        grid=(grid_n,),
        in_specs=[
            pl.BlockSpec((1, BR, H), lambda i: (0, i, 0)),
            pl.BlockSpec((1, BR, H), lambda i: (1, i, 0)),
            pl.BlockSpec((BR, H), lambda i: (i, 0)),
            pl.BlockSpec((BR, H), lambda i: (i, 0)),
            pl.BlockSpec((1, H), lambda i: (0, 0)),
            pl.BlockSpec((1, 1, BR), lambda i: (i, 0, 0)),
            pl.BlockSpec((H, C), lambda i: (0, 0)),
            pl.BlockSpec((1, C), lambda i: (0, 0)),
        ],
        out_specs=pl.BlockSpec((G, C), lambda i: (0, 0)),
        out_shape=jax.ShapeDtypeStruct((G, C), f32),
        scratch_shapes=[
            pltpu.VMEM((G, H), f32),
            pltpu.VMEM((G, H), f32),
        ],
    )(acc2, acc2, hs2, dis_b, b2.reshape(1, H), batch.reshape(grid_n, 1, BR),
      Wc, bc.reshape(1, C))
    return out


# speedup vs baseline: 1.5528x; 1.2600x over previous
import functools

import jax
import jax.numpy as jnp
from jax import lax
from jax.experimental import pallas as pl
from jax.experimental.pallas import tpu as pltpu
from jax.experimental.pallas import tpu_sc as plsc

NC = 2
NS = 16
NW = NC * NS
CW = 128
G = 64

_MESH = plsc.VectorSubcoreMesh(core_axis_name="c", subcore_axis_name="s")


def _deg_body(n0c, n1c, tc, e0, e1f, degr, dst_hbm, dstt_hbm, ones_hbm,
              zeros_hbm, out_hbm, dst_v, ones_v, zbuf_v, deg_s, sem):
    c = lax.axis_index("c")
    s = lax.axis_index("s")
    dpt = degr // NS
    base = pl.multiple_of(jnp.where(c == 0, s * e0, NS * e0 + s * e1f), 8)
    n = jnp.where(c == 0, n0c, n1c)

    def fill(j, carry):
        pltpu.async_copy(dst_hbm.at[pl.ds(base + j * CW, CW)], dst_v.at[j],
                         sem)
        return carry

    lax.fori_loop(0, n, fill, 0)
    pltpu.sync_copy(ones_hbm, ones_v)
    pltpu.sync_copy(zeros_hbm, zbuf_v)
    pltpu.sync_copy(zbuf_v, deg_s.at[pl.ds(s * dpt, dpt)])
    plsc.subcore_barrier()

    def drain(j, carry):
        pltpu.make_async_copy(dst_hbm.at[pl.ds(0, CW)], dst_v.at[0],
                              sem).wait()
        return carry

    lax.fori_loop(0, n, drain, 0)

    def body(j, carry):
        pltpu.sync_copy(ones_v, deg_s.at[dst_v.at[j]], add=True)
        return carry

    lax.fori_loop(0, n, body, 0)

    @pl.when(jnp.logical_and(c == 1, s < tc))
    def _():
        pltpu.sync_copy(dstt_hbm.at[s], dst_v.at[0])
        pltpu.sync_copy(ones_v, deg_s.at[dst_v.at[0]], add=True)

    plsc.subcore_barrier()
    pltpu.sync_copy(deg_s.at[pl.ds(s * dpt, dpt)], zbuf_v)
    pltpu.sync_copy(zbuf_v, out_hbm.at[pl.ds(c * degr + s * dpt, dpt)])


def _agg_body(n0c, n1c, tc, e0, e1f, accr, hs_hbm, src_hbm, dst_hbm,
              srct_hbm, dstt_hbm, zeros_hbm, out_hbm,
              srcidx_v, dstidx_v, rows_v, zbuf_v, acc_s,
              sem_ia, sem_ib, sem_ra, sem_rb):
    c = lax.axis_index("c")
    s = lax.axis_index("s")
    rpt = accr // NS
    rcw = rpt // CW
    zr = zbuf_v.shape[0]
    base = pl.multiple_of(jnp.where(c == 0, s * e0, NS * e0 + s * e1f), 8)

    isems = (sem_ia, sem_ib)
    rsems = (sem_ra, sem_rb)

    def idxfire(j, b):
        pltpu.async_copy(src_hbm.at[pl.ds(base + j * CW, CW)],
                         srcidx_v.at[b], isems[b])
        pltpu.async_copy(dst_hbm.at[pl.ds(base + j * CW, CW)],
                         dstidx_v.at[b], isems[b])

    def idxwait(b):
        pltpu.make_async_copy(src_hbm.at[pl.ds(0, CW)], srcidx_v.at[b],
                              isems[b]).wait()
        pltpu.make_async_copy(dst_hbm.at[pl.ds(0, CW)], dstidx_v.at[b],
                              isems[b]).wait()

    def rowfire(b):
        pltpu.async_copy(hs_hbm.at[srcidx_v.at[b]], rows_v.at[b], rsems[b])

    def rowwait(b):
        pltpu.make_async_copy(hs_hbm.at[pl.ds(0, CW)], rows_v.at[b],
                              rsems[b]).wait()

    def scatter(b):
        pltpu.sync_copy(rows_v.at[b], acc_s.at[dstidx_v.at[b]], add=True)

    def pipeline(nchunks):
        def step(j, b):
            @pl.when(j + 1 < nchunks)
            def _():
                idxwait(1 - b)
                rowfire(1 - b)

            rowwait(b)
            scatter(b)

            @pl.when(j + 2 < nchunks)
            def _():
                idxfire(j + 2, b)

        def body(p, carry):
            step(2 * p, 0)
            step(2 * p + 1, 1)
            return carry

        lax.fori_loop(0, nchunks // 2, body, 0)

    idxfire(0, 0)
    idxwait(0)
    rowfire(0)
    idxfire(1, 1)
    pltpu.sync_copy(zeros_hbm, zbuf_v)
    for k in range(rpt // zr):
        pltpu.sync_copy(zbuf_v, acc_s.at[pl.ds(s * rpt + k * zr, zr)])
    plsc.subcore_barrier()

    @pl.when(c == 0)
    def _():
        pipeline(n0c)

    @pl.when(c == 1)
    def _():
        pipeline(n1c)

    @pl.when(jnp.logical_and(c == 1, s < tc))
    def _():
        pltpu.sync_copy(srct_hbm.at[s], srcidx_v.at[0])
        pltpu.sync_copy(dstt_hbm.at[s], dstidx_v.at[0])
        pltpu.sync_copy(hs_hbm.at[srcidx_v.at[0]], rows_v.at[0])
        pltpu.sync_copy(rows_v.at[0], acc_s.at[dstidx_v.at[0]], add=True)

    plsc.subcore_barrier()
    for k in range(rcw):
        b = k % 2
        if k >= 2:
            pltpu.make_async_copy(rows_v.at[b], out_hbm.at[c, pl.ds(0, CW)],
                                  isems[b]).wait()
        pltpu.sync_copy(acc_s.at[pl.ds(s * rpt + k * CW, CW)], rows_v.at[b])
        pltpu.async_copy(rows_v.at[b],
                         out_hbm.at[c, pl.ds(s * rpt + k * CW, CW)], isems[b])
    for k in range(max(rcw - 2, 0), rcw):
        b = k % 2
        pltpu.make_async_copy(rows_v.at[b], out_hbm.at[c, pl.ds(0, CW)],
                              isems[b]).wait()


def _dis_tc(degT_ref, out_ref):
    d = jnp.sum(degT_ref[...], axis=1, keepdims=True) + 1.0
    dis = lax.rsqrt(d)
    out_ref[...] = jnp.broadcast_to(dis, out_ref.shape)


def _mm_scale_tc(x_ref, w_ref, dis_ref, out_ref):
    h = jnp.dot(x_ref[...], w_ref[...], preferred_element_type=jnp.float32)
    out_ref[...] = h * dis_ref[...]


def _mid_tc(a0_ref, a1_ref, hs_ref, dis_ref, b_ref, w_ref, out_ref):
    dis = dis_ref[...]
    t = (a0_ref[0] + a1_ref[0] + hs_ref[...]) * dis + b_ref[...]
    t = jnp.where(t >= 0, t, 0.2 * t)
    out_ref[...] = jnp.dot(t, w_ref[...],
                           preferred_element_type=jnp.float32) * dis


def _final_tc(a0_ref, a1_ref, hs_ref, dis_ref, b_ref, batch_ref, wc_ref,
              bc_ref, out_ref, sums, cnts):
    i = pl.program_id(0)
    n = pl.num_programs(0)
    dis = dis_ref[...]
    t = (a0_ref[0] + a1_ref[0] + hs_ref[...]) * dis + b_ref[...]
    t = jnp.where(t >= 0, t, 0.2 * t)
    rows = t.shape[0]
    oh = (lax.broadcasted_iota(jnp.int32, (G, rows), 0)
          == batch_ref[0]).astype(jnp.float32)

    @pl.when(i == 0)
    def _():
        sums[...] = jnp.zeros_like(sums)
        cnts[...] = jnp.zeros_like(cnts)

    sums[...] += jnp.dot(oh, t, preferred_element_type=jnp.float32)
    cnts[...] += jnp.broadcast_to(
        jnp.sum(oh, axis=1, keepdims=True), cnts.shape)

    @pl.when(i == n - 1)
    def _():
        pooled = sums[...] / jnp.maximum(cnts[...], 1.0)
        out_ref[...] = jnp.dot(pooled, wc_ref[...],
                               preferred_element_type=jnp.float32) + bc_ref[...]


def _ceil_to(a, m):
    return -(-a // m) * m


def kernel(x, edge_index, batch, W1, b1, W2, b2, Wc, bc):
    N, D = x.shape
    E = edge_index.shape[1]
    H = W1.shape[1]
    C = Wc.shape[1]
    f32 = jnp.float32

    accr = _ceil_to(N + 1, NS * CW)

    e_pt = E // NS
    e0 = (int(round(0.59 * e_pt)) // (2 * CW)) * (2 * CW)
    n0c = e0 // CW
    e1f = ((e_pt - e0) // (2 * CW)) * (2 * CW)
    n1c = e1f // CW
    tail = E - NS * (e0 + e1f)
    tc = tail // CW

    src_flat = edge_index[0]
    dst_flat = edge_index[1]
    tb = NS * (e0 + e1f)
    src_t = src_flat[tb:].reshape(tc, CW)
    dst_t = dst_flat[tb:].reshape(tc, CW)

    zeros1 = jnp.zeros((accr // NS,), f32)
    zeros2 = jnp.zeros((64, H), f32)
    ones1 = jnp.ones((CW,), f32)

    deg_kernel = pl.kernel(
        functools.partial(_deg_body, n0c, n1c, tc, e0, e1f, accr),
        out_type=jax.ShapeDtypeStruct((NC * accr,), f32),
        mesh=_MESH,
        scratch_types=[
            pltpu.VMEM((n0c, CW), jnp.int32),
            pltpu.VMEM((CW,), f32),
            pltpu.VMEM((accr // NS,), f32),
            pltpu.VMEM_SHARED((accr,), f32),
            pltpu.SemaphoreType.DMA,
        ],
    )
    degp = deg_kernel(dst_flat, dst_t, ones1, zeros1)

    agg_call = pl.kernel(
        functools.partial(_agg_body, n0c, n1c, tc, e0, e1f, accr),
        out_type=jax.ShapeDtypeStruct((NC, accr, H), f32),
        mesh=_MESH,
        scratch_types=[
            pltpu.VMEM((2, CW), jnp.int32),
            pltpu.VMEM((2, CW), jnp.int32),
            pltpu.VMEM((2, CW, H), f32),
            pltpu.VMEM((64, H), f32),
            pltpu.VMEM_SHARED((accr, H), f32),
            pltpu.SemaphoreType.DMA,
            pltpu.SemaphoreType.DMA,
            pltpu.SemaphoreType.DMA,
            pltpu.SemaphoreType.DMA,
        ],
    )

    degT = jnp.transpose(degp.reshape(NC, accr))
    bn = accr // 8
    dis_b = pl.pallas_call(
        _dis_tc,
        grid=(8,),
        in_specs=[pl.BlockSpec((bn, NC), lambda i: (i, 0))],
        out_specs=pl.BlockSpec((bn, H), lambda i: (i, 0)),
        out_shape=jax.ShapeDtypeStruct((accr, H), f32),
    )(degT)

    BR = 1000
    grid_n = N // BR

    hs1 = pl.pallas_call(
        _mm_scale_tc,
        grid=(grid_n,),
        in_specs=[
            pl.BlockSpec((BR, D), lambda i: (i, 0)),
            pl.BlockSpec((D, H), lambda i: (0, 0)),
            pl.BlockSpec((BR, H), lambda i: (i, 0)),
        ],
        out_specs=pl.BlockSpec((BR, H), lambda i: (i, 0)),
        out_shape=jax.ShapeDtypeStruct((N, H), f32),
    )(x, W1, dis_b)

    acc1 = agg_call(hs1, src_flat, dst_flat, src_t, dst_t, zeros2)

    hs2 = pl.pallas_call(
        _mid_tc,
        grid=(grid_n,),
        in_specs=[
            pl.BlockSpec((1, BR, H), lambda i: (0, i, 0)),
            pl.BlockSpec((1, BR, H), lambda i: (1, i, 0)),
            pl.BlockSpec((BR, H), lambda i: (i, 0)),
            pl.BlockSpec((BR, H), lambda i: (i, 0)),
            pl.BlockSpec((1, H), lambda i: (0, 0)),
            pl.BlockSpec((H, H), lambda i: (0, 0)),
        ],
        out_specs=pl.BlockSpec((BR, H), lambda i: (i, 0)),
        out_shape=jax.ShapeDtypeStruct((N, H), f32),
    )(acc1, acc1, hs1, dis_b, b1.reshape(1, H), W2)

    acc2 = agg_call(hs2, src_flat, dst_flat, src_t, dst_t, zeros2)

    out = pl.pallas_call(
        _final_tc,
        grid=(grid_n,),
        in_specs=[
            pl.BlockSpec((1, BR, H), lambda i: (0, i, 0)),
            pl.BlockSpec((1, BR, H), lambda i: (1, i, 0)),
            pl.BlockSpec((BR, H), lambda i: (i, 0)),
            pl.BlockSpec((BR, H), lambda i: (i, 0)),
            pl.BlockSpec((1, H), lambda i: (0, 0)),
            pl.BlockSpec((1, 1, BR), lambda i: (i, 0, 0)),
            pl.BlockSpec((H, C), lambda i: (0, 0)),
            pl.BlockSpec((1, C), lambda i: (0, 0)),
        ],
        out_specs=pl.BlockSpec((G, C), lambda i: (0, 0)),
        out_shape=jax.ShapeDtypeStruct((G, C), f32),
        scratch_shapes=[
            pltpu.VMEM((G, H), f32),
            pltpu.VMEM((G, H), f32),
        ],
    )(acc2, acc2, hs2, dis_b, b2.reshape(1, H), batch.reshape(grid_n, 1, BR),
      Wc, bc.reshape(1, C))
    return out


# speedup vs baseline: 1.6715x; 1.0764x over previous
import functools

import jax
import jax.numpy as jnp
from jax import lax
from jax.experimental import pallas as pl
from jax.experimental.pallas import tpu as pltpu
from jax.experimental.pallas import tpu_sc as plsc

NC = 2
NS = 16
NW = NC * NS
CW = 128
G = 64

_MESH = plsc.VectorSubcoreMesh(core_axis_name="c", subcore_axis_name="s")


def _deg_body(n0c, n1c, tc, e0, e1f, degr, dst_hbm, dstt_hbm, ones_hbm,
              zeros_hbm, out_hbm, dst_v, ones_v, zbuf_v, deg_s, sem):
    c = lax.axis_index("c")
    s = lax.axis_index("s")
    dpt = degr // NS
    base = pl.multiple_of(jnp.where(c == 0, s * e0, NS * e0 + s * e1f), 8)
    n = jnp.where(c == 0, n0c, n1c)

    def fill(j, carry):
        pltpu.async_copy(dst_hbm.at[pl.ds(base + j * CW, CW)], dst_v.at[j],
                         sem)
        return carry

    lax.fori_loop(0, n, fill, 0)
    pltpu.sync_copy(ones_hbm, ones_v)
    pltpu.sync_copy(zeros_hbm, zbuf_v)
    pltpu.sync_copy(zbuf_v, deg_s.at[pl.ds(s * dpt, dpt)])
    plsc.subcore_barrier()

    def drain(j, carry):
        pltpu.make_async_copy(dst_hbm.at[pl.ds(0, CW)], dst_v.at[0],
                              sem).wait()
        return carry

    lax.fori_loop(0, n, drain, 0)

    def body(j, carry):
        pltpu.sync_copy(ones_v, deg_s.at[dst_v.at[j]], add=True)
        return carry

    lax.fori_loop(0, n, body, 0)

    @pl.when(jnp.logical_and(c == 1, s < tc))
    def _():
        pltpu.sync_copy(dstt_hbm.at[s], dst_v.at[0])
        pltpu.sync_copy(ones_v, deg_s.at[dst_v.at[0]], add=True)

    plsc.subcore_barrier()
    pltpu.sync_copy(deg_s.at[pl.ds(s * dpt, dpt)], zbuf_v)
    pltpu.sync_copy(zbuf_v, out_hbm.at[pl.ds(c * degr + s * dpt, dpt)])


def _agg_body(n0c, n1c, tc, e0, e1f, accr, hs_hbm, src_hbm, dst_hbm,
              srct_hbm, dstt_hbm, zeros_hbm, out_hbm,
              srcidx_v, dstidx_v, rows_v, zbuf_v, acc_s,
              sem_ia, sem_ib, sem_ra, sem_rb):
    c = lax.axis_index("c")
    s = lax.axis_index("s")
    rpt = accr // NS
    rcw = rpt // CW
    zr = zbuf_v.shape[0]
    base = pl.multiple_of(jnp.where(c == 0, s * e0, NS * e0 + s * e1f), 8)

    isems = (sem_ia, sem_ib)
    rsems = (sem_ra, sem_rb)

    def idxfire(j, b):
        pltpu.async_copy(src_hbm.at[pl.ds(base + j * CW, CW)],
                         srcidx_v.at[b], isems[b])
        pltpu.async_copy(dst_hbm.at[pl.ds(base + j * CW, CW)],
                         dstidx_v.at[b], isems[b])

    def idxwait(b):
        pltpu.make_async_copy(src_hbm.at[pl.ds(0, CW)], srcidx_v.at[b],
                              isems[b]).wait()
        pltpu.make_async_copy(dst_hbm.at[pl.ds(0, CW)], dstidx_v.at[b],
                              isems[b]).wait()

    def rowfire(b):
        pltpu.async_copy(hs_hbm.at[srcidx_v.at[b]], rows_v.at[b], rsems[b])

    def rowwait(b):
        pltpu.make_async_copy(hs_hbm.at[pl.ds(0, CW)], rows_v.at[b],
                              rsems[b]).wait()

    def scatter(b):
        pltpu.sync_copy(rows_v.at[b], acc_s.at[dstidx_v.at[b]], add=True)

    def pipeline(nchunks):
        def step(j, b):
            @pl.when(j + 1 < nchunks)
            def _():
                idxwait(1 - b)
                rowfire(1 - b)

            rowwait(b)
            scatter(b)

            @pl.when(j + 2 < nchunks)
            def _():
                idxfire(j + 2, b)

        def body(p, carry):
            step(2 * p, 0)
            step(2 * p + 1, 1)
            return carry

        lax.fori_loop(0, nchunks // 2, body, 0)

    idxfire(0, 0)
    idxwait(0)
    rowfire(0)
    idxfire(1, 1)
    pltpu.sync_copy(zeros_hbm, zbuf_v)
    for k in range(rpt // zr):
        pltpu.sync_copy(zbuf_v, acc_s.at[pl.ds(s * rpt + k * zr, zr)])
    plsc.subcore_barrier()

    @pl.when(c == 0)
    def _():
        pipeline(n0c)

    @pl.when(c == 1)
    def _():
        pipeline(n1c)

    @pl.when(jnp.logical_and(c == 1, s < tc))
    def _():
        pltpu.sync_copy(srct_hbm.at[s], srcidx_v.at[0])
        pltpu.sync_copy(dstt_hbm.at[s], dstidx_v.at[0])
        pltpu.sync_copy(hs_hbm.at[srcidx_v.at[0]], rows_v.at[0])
        pltpu.sync_copy(rows_v.at[0], acc_s.at[dstidx_v.at[0]], add=True)

    plsc.subcore_barrier()
    for k in range(rcw):
        b = k % 2
        if k >= 2:
            pltpu.make_async_copy(rows_v.at[b], out_hbm.at[c, pl.ds(0, CW)],
                                  isems[b]).wait()
        pltpu.sync_copy(acc_s.at[pl.ds(s * rpt + k * CW, CW)], rows_v.at[b])
        pltpu.async_copy(rows_v.at[b],
                         out_hbm.at[c, pl.ds(s * rpt + k * CW, CW)], isems[b])
    for k in range(max(rcw - 2, 0), rcw):
        b = k % 2
        pltpu.make_async_copy(rows_v.at[b], out_hbm.at[c, pl.ds(0, CW)],
                              isems[b]).wait()


def _dis_tc(degT_ref, out_ref):
    d = jnp.sum(degT_ref[...], axis=1, keepdims=True) + 1.0
    dis = lax.rsqrt(d)
    out_ref[...] = jnp.broadcast_to(dis, out_ref.shape)


def _mm_scale_tc(x_ref, w_ref, dis_ref, out_ref):
    h = jnp.dot(x_ref[...], w_ref[...], preferred_element_type=jnp.float32)
    out_ref[...] = h * dis_ref[...]


def _mid_tc(a0_ref, a1_ref, hs_ref, dis_ref, b_ref, w_ref, out_ref):
    dis = dis_ref[...]
    t = (a0_ref[0] + a1_ref[0] + hs_ref[...]) * dis + b_ref[...]
    t = jnp.where(t >= 0, t, 0.2 * t)
    out_ref[...] = jnp.dot(t, w_ref[...],
                           preferred_element_type=jnp.float32) * dis


def _final_tc(a0_ref, a1_ref, hs_ref, dis_ref, b_ref, batch_ref, wc_ref,
              bc_ref, out_ref, sums, cnts):
    i = pl.program_id(0)
    n = pl.num_programs(0)
    dis = dis_ref[...]
    t = (a0_ref[0] + a1_ref[0] + hs_ref[...]) * dis + b_ref[...]
    t = jnp.where(t >= 0, t, 0.2 * t)
    rows = t.shape[0]
    oh = (lax.broadcasted_iota(jnp.int32, (G, rows), 0)
          == batch_ref[0]).astype(jnp.float32)

    @pl.when(i == 0)
    def _():
        sums[...] = jnp.zeros_like(sums)
        cnts[...] = jnp.zeros_like(cnts)

    sums[...] += jnp.dot(oh, t, preferred_element_type=jnp.float32)
    cnts[...] += jnp.broadcast_to(
        jnp.sum(oh, axis=1, keepdims=True), cnts.shape)

    @pl.when(i == n - 1)
    def _():
        pooled = sums[...] / jnp.maximum(cnts[...], 1.0)
        out_ref[...] = jnp.dot(pooled, wc_ref[...],
                               preferred_element_type=jnp.float32) + bc_ref[...]


def _ceil_to(a, m):
    return -(-a // m) * m


def kernel(x, edge_index, batch, W1, b1, W2, b2, Wc, bc):
    N, D = x.shape
    E = edge_index.shape[1]
    H = W1.shape[1]
    C = Wc.shape[1]
    f32 = jnp.float32

    accr = _ceil_to(N + 1, NS * CW)

    e_pt = E // NS
    e0 = (int(round(0.53 * e_pt)) // (2 * CW)) * (2 * CW)
    n0c = e0 // CW
    e1f = ((e_pt - e0) // (2 * CW)) * (2 * CW)
    n1c = e1f // CW
    tail = E - NS * (e0 + e1f)
    tc = tail // CW

    src_flat = edge_index[0]
    dst_flat = edge_index[1]
    tb = NS * (e0 + e1f)
    src_t = src_flat[tb:].reshape(tc, CW)
    dst_t = dst_flat[tb:].reshape(tc, CW)

    zeros1 = jnp.zeros((accr // NS,), f32)
    zeros2 = jnp.zeros((64, H), f32)
    ones1 = jnp.ones((CW,), f32)

    deg_kernel = pl.kernel(
        functools.partial(_deg_body, n0c, n1c, tc, e0, e1f, accr),
        out_type=jax.ShapeDtypeStruct((NC * accr,), f32),
        mesh=_MESH,
        scratch_types=[
            pltpu.VMEM((n0c, CW), jnp.int32),
            pltpu.VMEM((CW,), f32),
            pltpu.VMEM((accr // NS,), f32),
            pltpu.VMEM_SHARED((accr,), f32),
            pltpu.SemaphoreType.DMA,
        ],
    )
    degp = deg_kernel(dst_flat, dst_t, ones1, zeros1)

    agg_call = pl.kernel(
        functools.partial(_agg_body, n0c, n1c, tc, e0, e1f, accr),
        out_type=jax.ShapeDtypeStruct((NC, accr, H), f32),
        mesh=_MESH,
        scratch_types=[
            pltpu.VMEM((2, CW), jnp.int32),
            pltpu.VMEM((2, CW), jnp.int32),
            pltpu.VMEM((2, CW, H), f32),
            pltpu.VMEM((64, H), f32),
            pltpu.VMEM_SHARED((accr, H), f32),
            pltpu.SemaphoreType.DMA,
            pltpu.SemaphoreType.DMA,
            pltpu.SemaphoreType.DMA,
            pltpu.SemaphoreType.DMA,
        ],
    )

    degT = jnp.transpose(degp.reshape(NC, accr))
    bn = accr // 8
    dis_b = pl.pallas_call(
        _dis_tc,
        grid=(8,),
        in_specs=[pl.BlockSpec((bn, NC), lambda i: (i, 0))],
        out_specs=pl.BlockSpec((bn, H), lambda i: (i, 0)),
        out_shape=jax.ShapeDtypeStruct((accr, H), f32),
    )(degT)

    BR = 1000
    grid_n = N // BR

    hs1 = pl.pallas_call(
        _mm_scale_tc,
        grid=(grid_n,),
        in_specs=[
            pl.BlockSpec((BR, D), lambda i: (i, 0)),
            pl.BlockSpec((D, H), lambda i: (0, 0)),
            pl.BlockSpec((BR, H), lambda i: (i, 0)),
        ],
        out_specs=pl.BlockSpec((BR, H), lambda i: (i, 0)),
        out_shape=jax.ShapeDtypeStruct((N, H), f32),
    )(x, W1, dis_b)

    acc1 = agg_call(hs1, src_flat, dst_flat, src_t, dst_t, zeros2)

    hs2 = pl.pallas_call(
        _mid_tc,
        grid=(grid_n,),
        in_specs=[
            pl.BlockSpec((1, BR, H), lambda i: (0, i, 0)),
            pl.BlockSpec((1, BR, H), lambda i: (1, i, 0)),
            pl.BlockSpec((BR, H), lambda i: (i, 0)),
            pl.BlockSpec((BR, H), lambda i: (i, 0)),
            pl.BlockSpec((1, H), lambda i: (0, 0)),
            pl.BlockSpec((H, H), lambda i: (0, 0)),
        ],
        out_specs=pl.BlockSpec((BR, H), lambda i: (i, 0)),
        out_shape=jax.ShapeDtypeStruct((N, H), f32),
    )(acc1, acc1, hs1, dis_b, b1.reshape(1, H), W2)

    acc2 = agg_call(hs2, src_flat, dst_flat, src_t, dst_t, zeros2)

    out = pl.pallas_call(
        _final_tc,
        grid=(grid_n,),
        in_specs=[
            pl.BlockSpec((1, BR, H), lambda i: (0, i, 0)),
            pl.BlockSpec((1, BR, H), lambda i: (1, i, 0)),
            pl.BlockSpec((BR, H), lambda i: (i, 0)),
            pl.BlockSpec((BR, H), lambda i: (i, 0)),
            pl.BlockSpec((1, H), lambda i: (0, 0)),
            pl.BlockSpec((1, 1, BR), lambda i: (i, 0, 0)),
            pl.BlockSpec((H, C), lambda i: (0, 0)),
            pl.BlockSpec((1, C), lambda i: (0, 0)),
        ],
        out_specs=pl.BlockSpec((G, C), lambda i: (0, 0)),
        out_shape=jax.ShapeDtypeStruct((G, C), f32),
        scratch_shapes=[
            pltpu.VMEM((G, H), f32),
            pltpu.VMEM((G, H), f32),
        ],
    )(acc2, acc2, hs2, dis_b, b2.reshape(1, H), batch.reshape(grid_n, 1, BR),
      Wc, bc.reshape(1, C))
    return out


# speedup vs baseline: 1.7447x; 1.0439x over previous
import functools

import jax
import jax.numpy as jnp
from jax import lax
from jax.experimental import pallas as pl
from jax.experimental.pallas import tpu as pltpu
from jax.experimental.pallas import tpu_sc as plsc

NC = 2
NS = 16
NW = NC * NS
CW = 128
G = 64

_MESH = plsc.VectorSubcoreMesh(core_axis_name="c", subcore_axis_name="s")


def _deg_body(n0c, n1c, tc, e0, e1f, dso, degr, edges_hbm, dstt_hbm, ones_hbm,
              zeros_hbm, out_hbm, dst_v, ones_v, zbuf_v, deg_s, sem):
    c = lax.axis_index("c")
    s = lax.axis_index("s")
    dpt = degr // NS
    base = pl.multiple_of(
        dso + jnp.where(c == 0, s * e0, NS * e0 + s * e1f), 8)
    n = jnp.where(c == 0, n0c, n1c)

    def fill(j, carry):
        pltpu.async_copy(edges_hbm.at[pl.ds(base + j * CW, CW)], dst_v.at[j],
                         sem)
        return carry

    lax.fori_loop(0, n, fill, 0)
    pltpu.sync_copy(ones_hbm, ones_v)
    pltpu.sync_copy(zeros_hbm, zbuf_v)
    pltpu.sync_copy(zbuf_v, deg_s.at[pl.ds(s * dpt, dpt)])
    plsc.subcore_barrier()

    def drain(j, carry):
        pltpu.make_async_copy(edges_hbm.at[pl.ds(0, CW)], dst_v.at[0],
                              sem).wait()
        return carry

    lax.fori_loop(0, n, drain, 0)

    def body(j, carry):
        pltpu.sync_copy(ones_v, deg_s.at[dst_v.at[j]], add=True)
        return carry

    lax.fori_loop(0, n, body, 0)

    @pl.when(jnp.logical_and(c == 1, s < tc))
    def _():
        pltpu.sync_copy(dstt_hbm.at[s], dst_v.at[0])
        pltpu.sync_copy(ones_v, deg_s.at[dst_v.at[0]], add=True)

    plsc.subcore_barrier()
    pltpu.sync_copy(deg_s.at[pl.ds(s * dpt, dpt)], zbuf_v)
    pltpu.sync_copy(zbuf_v, out_hbm.at[pl.ds(c * degr + s * dpt, dpt)])


def _agg_body(n0c, n1c, tc, e0, e1f, dso, accr, hs_hbm, edges_hbm,
              srct_hbm, dstt_hbm, zeros_hbm, out_hbm,
              srcidx_v, dstidx_v, rows_v, zbuf_v, acc_s,
              sem_ia, sem_ib, sem_ra, sem_rb):
    c = lax.axis_index("c")
    s = lax.axis_index("s")
    rpt = accr // NS
    rcw = rpt // CW
    zr = zbuf_v.shape[0]
    base = pl.multiple_of(jnp.where(c == 0, s * e0, NS * e0 + s * e1f), 8)

    isems = (sem_ia, sem_ib)
    rsems = (sem_ra, sem_rb)

    def idxfire(j, b):
        pltpu.async_copy(edges_hbm.at[pl.ds(base + j * CW, CW)],
                         srcidx_v.at[b], isems[b])
        pltpu.async_copy(edges_hbm.at[pl.ds(dso + base + j * CW, CW)],
                         dstidx_v.at[b], isems[b])

    def idxwait(b):
        pltpu.make_async_copy(edges_hbm.at[pl.ds(0, CW)], srcidx_v.at[b],
                              isems[b]).wait()
        pltpu.make_async_copy(edges_hbm.at[pl.ds(0, CW)], dstidx_v.at[b],
                              isems[b]).wait()

    def rowfire(b):
        pltpu.async_copy(hs_hbm.at[srcidx_v.at[b]], rows_v.at[b], rsems[b])

    def rowwait(b):
        pltpu.make_async_copy(hs_hbm.at[pl.ds(0, CW)], rows_v.at[b],
                              rsems[b]).wait()

    def scatter(b):
        pltpu.sync_copy(rows_v.at[b], acc_s.at[dstidx_v.at[b]], add=True)

    def pipeline(nchunks):
        def step(j, b):
            @pl.when(j + 1 < nchunks)
            def _():
                idxwait(1 - b)
                rowfire(1 - b)

            rowwait(b)
            scatter(b)

            @pl.when(j + 2 < nchunks)
            def _():
                idxfire(j + 2, b)

        def body(p, carry):
            step(2 * p, 0)
            step(2 * p + 1, 1)
            return carry

        lax.fori_loop(0, nchunks // 2, body, 0)

    idxfire(0, 0)
    idxwait(0)
    rowfire(0)
    idxfire(1, 1)
    pltpu.sync_copy(zeros_hbm, zbuf_v)
    for k in range(rpt // zr):
        pltpu.sync_copy(zbuf_v, acc_s.at[pl.ds(s * rpt + k * zr, zr)])
    plsc.subcore_barrier()

    @pl.when(c == 0)
    def _():
        pipeline(n0c)

    @pl.when(c == 1)
    def _():
        pipeline(n1c)

    @pl.when(jnp.logical_and(c == 1, s < tc))
    def _():
        pltpu.sync_copy(srct_hbm.at[s], srcidx_v.at[0])
        pltpu.sync_copy(dstt_hbm.at[s], dstidx_v.at[0])
        pltpu.sync_copy(hs_hbm.at[srcidx_v.at[0]], rows_v.at[0])
        pltpu.sync_copy(rows_v.at[0], acc_s.at[dstidx_v.at[0]], add=True)

    plsc.subcore_barrier()
    for k in range(rcw):
        b = k % 2
        if k >= 2:
            pltpu.make_async_copy(rows_v.at[b], out_hbm.at[c, pl.ds(0, CW)],
                                  isems[b]).wait()
        pltpu.sync_copy(acc_s.at[pl.ds(s * rpt + k * CW, CW)], rows_v.at[b])
        pltpu.async_copy(rows_v.at[b],
                         out_hbm.at[c, pl.ds(s * rpt + k * CW, CW)], isems[b])
    for k in range(max(rcw - 2, 0), rcw):
        b = k % 2
        pltpu.make_async_copy(rows_v.at[b], out_hbm.at[c, pl.ds(0, CW)],
                              isems[b]).wait()


def _mm_tc(x_ref, w_ref, out_ref):
    out_ref[...] = jnp.dot(x_ref[...], w_ref[...],
                           preferred_element_type=jnp.float32)


def _dis_scale_tc(degT_ref, h_ref, dis_ref, hs_ref):
    d = jnp.sum(degT_ref[...], axis=1, keepdims=True) + 1.0
    dis = jnp.broadcast_to(lax.rsqrt(d), dis_ref.shape)
    dis_ref[...] = dis
    hs_ref[...] = h_ref[...] * dis


def _mid_tc(a0_ref, a1_ref, hs_ref, dis_ref, b_ref, w_ref, out_ref):
    dis = dis_ref[...]
    t = (a0_ref[0] + a1_ref[0] + hs_ref[...]) * dis + b_ref[...]
    t = jnp.where(t >= 0, t, 0.2 * t)
    out_ref[...] = jnp.dot(t, w_ref[...],
                           preferred_element_type=jnp.float32) * dis


def _final_tc(a0_ref, a1_ref, hs_ref, dis_ref, b_ref, batch_ref, wc_ref,
              bc_ref, out_ref, sums, cnts):
    i = pl.program_id(0)
    n = pl.num_programs(0)
    dis = dis_ref[...]
    t = (a0_ref[0] + a1_ref[0] + hs_ref[...]) * dis + b_ref[...]
    t = jnp.where(t >= 0, t, 0.2 * t)
    rows = t.shape[0]
    oh = (lax.broadcasted_iota(jnp.int32, (G, rows), 0)
          == batch_ref[0]).astype(jnp.float32)

    @pl.when(i == 0)
    def _():
        sums[...] = jnp.zeros_like(sums)
        cnts[...] = jnp.zeros_like(cnts)

    sums[...] += jnp.dot(oh, t, preferred_element_type=jnp.float32)
    cnts[...] += jnp.broadcast_to(
        jnp.sum(oh, axis=1, keepdims=True), cnts.shape)

    @pl.when(i == n - 1)
    def _():
        pooled = sums[...] / jnp.maximum(cnts[...], 1.0)
        out_ref[...] = jnp.dot(pooled, wc_ref[...],
                               preferred_element_type=jnp.float32) + bc_ref[...]


def _ceil_to(a, m):
    return -(-a // m) * m


def kernel(x, edge_index, batch, W1, b1, W2, b2, Wc, bc):
    N, D = x.shape
    E = edge_index.shape[1]
    H = W1.shape[1]
    C = Wc.shape[1]
    f32 = jnp.float32

    accr = _ceil_to(N + 1, NS * CW)

    e_pt = E // NS
    e0 = (int(round(0.53 * e_pt)) // (2 * CW)) * (2 * CW)
    n0c = e0 // CW
    e1f = ((e_pt - e0) // (2 * CW)) * (2 * CW)
    n1c = e1f // CW
    tail = E - NS * (e0 + e1f)
    tc = tail // CW

    edges1d = edge_index.reshape(2 * E)
    tb = NS * (e0 + e1f)
    src_t = edges1d[tb:tb + tail].reshape(tc, CW)
    dst_t = edges1d[E + tb:E + tb + tail].reshape(tc, CW)

    zeros1 = jnp.zeros((accr // NS,), f32)
    zeros2 = jnp.zeros((64, H), f32)
    ones1 = jnp.ones((CW,), f32)

    deg_kernel = pl.kernel(
        functools.partial(_deg_body, n0c, n1c, tc, e0, e1f, E, accr),
        out_type=jax.ShapeDtypeStruct((NC * accr,), f32),
        mesh=_MESH,
        scratch_types=[
            pltpu.VMEM((n0c, CW), jnp.int32),
            pltpu.VMEM((CW,), f32),
            pltpu.VMEM((accr // NS,), f32),
            pltpu.VMEM_SHARED((accr,), f32),
            pltpu.SemaphoreType.DMA,
        ],
    )
    degp = deg_kernel(edges1d, dst_t, ones1, zeros1)

    agg_call = pl.kernel(
        functools.partial(_agg_body, n0c, n1c, tc, e0, e1f, E, accr),
        out_type=jax.ShapeDtypeStruct((NC, accr, H), f32),
        mesh=_MESH,
        scratch_types=[
            pltpu.VMEM((2, CW), jnp.int32),
            pltpu.VMEM((2, CW), jnp.int32),
            pltpu.VMEM((2, CW, H), f32),
            pltpu.VMEM((64, H), f32),
            pltpu.VMEM_SHARED((accr, H), f32),
            pltpu.SemaphoreType.DMA,
            pltpu.SemaphoreType.DMA,
            pltpu.SemaphoreType.DMA,
            pltpu.SemaphoreType.DMA,
        ],
    )

    BR = 1000
    grid_n = N // BR

    h1 = pl.pallas_call(
        _mm_tc,
        grid=(grid_n,),
        in_specs=[
            pl.BlockSpec((BR, D), lambda i: (i, 0)),
            pl.BlockSpec((D, H), lambda i: (0, 0)),
        ],
        out_specs=pl.BlockSpec((BR, H), lambda i: (i, 0)),
        out_shape=jax.ShapeDtypeStruct((N, H), f32),
    )(x, W1)

    degT = jnp.transpose(degp.reshape(NC, accr))
    dis_b, hs1 = pl.pallas_call(
        _dis_scale_tc,
        grid=(grid_n,),
        in_specs=[
            pl.BlockSpec((BR, NC), lambda i: (i, 0)),
            pl.BlockSpec((BR, H), lambda i: (i, 0)),
        ],
        out_specs=[
            pl.BlockSpec((BR, H), lambda i: (i, 0)),
            pl.BlockSpec((BR, H), lambda i: (i, 0)),
        ],
        out_shape=[
            jax.ShapeDtypeStruct((N, H), f32),
            jax.ShapeDtypeStruct((N, H), f32),
        ],
    )(degT, h1)

    acc1 = agg_call(hs1, edges1d, src_t, dst_t, zeros2)

    hs2 = pl.pallas_call(
        _mid_tc,
        grid=(grid_n,),
        in_specs=[
            pl.BlockSpec((1, BR, H), lambda i: (0, i, 0)),
            pl.BlockSpec((1, BR, H), lambda i: (1, i, 0)),
            pl.BlockSpec((BR, H), lambda i: (i, 0)),
            pl.BlockSpec((BR, H), lambda i: (i, 0)),
            pl.BlockSpec((1, H), lambda i: (0, 0)),
            pl.BlockSpec((H, H), lambda i: (0, 0)),
        ],
        out_specs=pl.BlockSpec((BR, H), lambda i: (i, 0)),
        out_shape=jax.ShapeDtypeStruct((N, H), f32),
    )(acc1, acc1, hs1, dis_b, b1.reshape(1, H), W2)

    acc2 = agg_call(hs2, edges1d, src_t, dst_t, zeros2)

    out = pl.pallas_call(
        _final_tc,
        grid=(grid_n,),
        in_specs=[
            pl.BlockSpec((1, BR, H), lambda i: (0, i, 0)),
            pl.BlockSpec((1, BR, H), lambda i: (1, i, 0)),
            pl.BlockSpec((BR, H), lambda i: (i, 0)),
            pl.BlockSpec((BR, H), lambda i: (i, 0)),
            pl.BlockSpec((1, H), lambda i: (0, 0)),
            pl.BlockSpec((1, 1, BR), lambda i: (i, 0, 0)),
            pl.BlockSpec((H, C), lambda i: (0, 0)),
            pl.BlockSpec((1, C), lambda i: (0, 0)),
        ],
        out_specs=pl.BlockSpec((G, C), lambda i: (0, 0)),
        out_shape=jax.ShapeDtypeStruct((G, C), f32),
        scratch_shapes=[
            pltpu.VMEM((G, H), f32),
            pltpu.VMEM((G, H), f32),
        ],
    )(acc2, acc2, hs2, dis_b, b2.reshape(1, H), batch.reshape(grid_n, 1, BR),
      Wc, bc.reshape(1, C))
    return out


# speedup vs baseline: 1.7745x; 1.0170x over previous
import functools

import jax
import jax.numpy as jnp
from jax import lax
from jax.experimental import pallas as pl
from jax.experimental.pallas import tpu as pltpu
from jax.experimental.pallas import tpu_sc as plsc

NC = 2
NS = 16
NW = NC * NS
CW = 128
G = 64

_MESH = plsc.VectorSubcoreMesh(core_axis_name="c", subcore_axis_name="s")


def _deg_body(n0c, n1c, tc, e0, e1f, dso, degr, edges_hbm, dstt_hbm, ones_hbm,
              zeros_hbm, out_hbm, dst_v, ones_v, zbuf_v, deg_s, sem):
    c = lax.axis_index("c")
    s = lax.axis_index("s")
    dpt = degr // NS
    base = pl.multiple_of(
        dso + jnp.where(c == 0, s * e0, NS * e0 + s * e1f), 8)
    n = jnp.where(c == 0, n0c, n1c)

    def fill(j, carry):
        pltpu.async_copy(edges_hbm.at[pl.ds(base + j * CW, CW)], dst_v.at[j],
                         sem)
        return carry

    lax.fori_loop(0, n, fill, 0)
    pltpu.sync_copy(ones_hbm, ones_v)
    pltpu.sync_copy(zeros_hbm, zbuf_v)
    pltpu.sync_copy(zbuf_v, deg_s.at[pl.ds(s * dpt, dpt)])
    plsc.subcore_barrier()

    def drain(j, carry):
        pltpu.make_async_copy(edges_hbm.at[pl.ds(0, CW)], dst_v.at[0],
                              sem).wait()
        return carry

    lax.fori_loop(0, n, drain, 0)

    def body(j, carry):
        pltpu.sync_copy(ones_v, deg_s.at[dst_v.at[j]], add=True)
        return carry

    lax.fori_loop(0, n, body, 0)

    @pl.when(jnp.logical_and(c == 1, s < tc))
    def _():
        pltpu.sync_copy(dstt_hbm.at[s], dst_v.at[0])
        pltpu.sync_copy(ones_v, deg_s.at[dst_v.at[0]], add=True)

    plsc.subcore_barrier()
    pltpu.sync_copy(deg_s.at[pl.ds(s * dpt, dpt)], zbuf_v)
    pltpu.sync_copy(zbuf_v, out_hbm.at[pl.ds(c * degr + s * dpt, dpt)])


def _agg_body(n0c, n1c, tc, e0, e1f, dso, accr, hs_hbm, edges_hbm,
              srct_hbm, dstt_hbm, zeros_hbm, out_hbm,
              srcidx_v, dstidx_v, rows_v, zbuf_v, acc_s,
              sem_ia, sem_ib, sem_ra, sem_rb):
    c = lax.axis_index("c")
    s = lax.axis_index("s")
    rpt = accr // NS
    rcw = rpt // CW
    zr = zbuf_v.shape[0]
    base = pl.multiple_of(jnp.where(c == 0, s * e0, NS * e0 + s * e1f), 8)

    isems = (sem_ia, sem_ib)
    rsems = (sem_ra, sem_rb)

    def idxfire(j, b):
        pltpu.async_copy(edges_hbm.at[pl.ds(base + j * CW, CW)],
                         srcidx_v.at[b], isems[b])
        pltpu.async_copy(edges_hbm.at[pl.ds(dso + base + j * CW, CW)],
                         dstidx_v.at[b], isems[b])

    def idxwait(b):
        pltpu.make_async_copy(edges_hbm.at[pl.ds(0, CW)], srcidx_v.at[b],
                              isems[b]).wait()
        pltpu.make_async_copy(edges_hbm.at[pl.ds(0, CW)], dstidx_v.at[b],
                              isems[b]).wait()

    def rowfire(b):
        pltpu.async_copy(hs_hbm.at[srcidx_v.at[b]], rows_v.at[b], rsems[b])

    def rowwait(b):
        pltpu.make_async_copy(hs_hbm.at[pl.ds(0, CW)], rows_v.at[b],
                              rsems[b]).wait()

    def scatter(b):
        pltpu.sync_copy(rows_v.at[b], acc_s.at[dstidx_v.at[b]], add=True)

    def pipeline(nchunks):
        def step(j, b):
            @pl.when(j + 1 < nchunks)
            def _():
                idxwait(1 - b)
                rowfire(1 - b)

            rowwait(b)
            scatter(b)

            @pl.when(j + 2 < nchunks)
            def _():
                idxfire(j + 2, b)

        def body(p, carry):
            step(2 * p, 0)
            step(2 * p + 1, 1)
            return carry

        lax.fori_loop(0, nchunks // 2, body, 0)

    idxfire(0, 0)
    idxwait(0)
    rowfire(0)
    idxfire(1, 1)
    pltpu.sync_copy(zeros_hbm, zbuf_v)
    for k in range(rpt // zr):
        pltpu.sync_copy(zbuf_v, acc_s.at[pl.ds(s * rpt + k * zr, zr)])
    plsc.subcore_barrier()

    @pl.when(c == 0)
    def _():
        pipeline(n0c)

    @pl.when(c == 1)
    def _():
        pipeline(n1c)

    @pl.when(jnp.logical_and(c == 1, s < tc))
    def _():
        pltpu.sync_copy(srct_hbm.at[s], srcidx_v.at[0])
        pltpu.sync_copy(dstt_hbm.at[s], dstidx_v.at[0])
        pltpu.sync_copy(hs_hbm.at[srcidx_v.at[0]], rows_v.at[0])
        pltpu.sync_copy(rows_v.at[0], acc_s.at[dstidx_v.at[0]], add=True)

    plsc.subcore_barrier()
    for k in range(rcw):
        b = k % 2
        if k >= 2:
            pltpu.make_async_copy(rows_v.at[b], out_hbm.at[c, pl.ds(0, CW)],
                                  isems[b]).wait()
        pltpu.sync_copy(acc_s.at[pl.ds(s * rpt + k * CW, CW)], rows_v.at[b])
        pltpu.async_copy(rows_v.at[b],
                         out_hbm.at[c, pl.ds(s * rpt + k * CW, CW)], isems[b])
    for k in range(max(rcw - 2, 0), rcw):
        b = k % 2
        pltpu.make_async_copy(rows_v.at[b], out_hbm.at[c, pl.ds(0, CW)],
                              isems[b]).wait()


def _mm_tc(x_ref, w_ref, out_ref):
    out_ref[...] = jnp.dot(x_ref[...], w_ref[...],
                           preferred_element_type=jnp.float32)


def _dis_scale_tc(degT_ref, h_ref, dis_ref, hs_ref):
    d = jnp.sum(degT_ref[...], axis=1, keepdims=True) + 1.0
    dis = jnp.broadcast_to(lax.rsqrt(d), dis_ref.shape)
    dis_ref[...] = dis
    hs_ref[...] = h_ref[...] * dis


def _mid_tc(a0_ref, a1_ref, hs_ref, dis_ref, b_ref, w_ref, out_ref):
    dis = dis_ref[...]
    t = (a0_ref[0] + a1_ref[0] + hs_ref[...]) * dis + b_ref[...]
    t = jnp.where(t >= 0, t, 0.2 * t)
    out_ref[...] = jnp.dot(t, w_ref[...],
                           preferred_element_type=jnp.float32) * dis


def _final_tc(a0_ref, a1_ref, hs_ref, dis_ref, b_ref, batch_ref, wc_ref,
              bc_ref, out_ref, sums, cnts):
    i = pl.program_id(0)
    n = pl.num_programs(0)
    dis = dis_ref[...]
    t = (a0_ref[0] + a1_ref[0] + hs_ref[...]) * dis + b_ref[...]
    t = jnp.where(t >= 0, t, 0.2 * t)
    rows = t.shape[0]
    oh = (lax.broadcasted_iota(jnp.int32, (G, rows), 0)
          == batch_ref[0]).astype(jnp.float32)

    @pl.when(i == 0)
    def _():
        sums[...] = jnp.zeros_like(sums)
        cnts[...] = jnp.zeros_like(cnts)

    sums[...] += jnp.dot(oh, t, preferred_element_type=jnp.float32)
    cnts[...] += jnp.broadcast_to(
        jnp.sum(oh, axis=1, keepdims=True), cnts.shape)

    @pl.when(i == n - 1)
    def _():
        pooled = sums[...] / jnp.maximum(cnts[...], 1.0)
        out_ref[...] = jnp.dot(pooled, wc_ref[...],
                               preferred_element_type=jnp.float32) + bc_ref[...]


def _ceil_to(a, m):
    return -(-a // m) * m


def kernel(x, edge_index, batch, W1, b1, W2, b2, Wc, bc):
    N, D = x.shape
    E = edge_index.shape[1]
    H = W1.shape[1]
    C = Wc.shape[1]
    f32 = jnp.float32

    accr = _ceil_to(N + 1, NS * CW)

    e_pt = E // NS
    e0 = (int(round(0.515 * e_pt)) // (2 * CW)) * (2 * CW)
    n0c = e0 // CW
    e1f = ((e_pt - e0) // (2 * CW)) * (2 * CW)
    n1c = e1f // CW
    tail = E - NS * (e0 + e1f)
    tc = tail // CW

    edges1d = edge_index.reshape(2 * E)
    tb = NS * (e0 + e1f)
    src_t = edges1d[tb:tb + tail].reshape(tc, CW)
    dst_t = edges1d[E + tb:E + tb + tail].reshape(tc, CW)

    zeros1 = jnp.zeros((accr // NS,), f32)
    zeros2 = jnp.zeros((64, H), f32)
    ones1 = jnp.ones((CW,), f32)

    deg_kernel = pl.kernel(
        functools.partial(_deg_body, n0c, n1c, tc, e0, e1f, E, accr),
        out_type=jax.ShapeDtypeStruct((NC * accr,), f32),
        mesh=_MESH,
        scratch_types=[
            pltpu.VMEM((n0c, CW), jnp.int32),
            pltpu.VMEM((CW,), f32),
            pltpu.VMEM((accr // NS,), f32),
            pltpu.VMEM_SHARED((accr,), f32),
            pltpu.SemaphoreType.DMA,
        ],
    )
    degp = deg_kernel(edges1d, dst_t, ones1, zeros1)

    agg_call = pl.kernel(
        functools.partial(_agg_body, n0c, n1c, tc, e0, e1f, E, accr),
        out_type=jax.ShapeDtypeStruct((NC, accr, H), f32),
        mesh=_MESH,
        scratch_types=[
            pltpu.VMEM((2, CW), jnp.int32),
            pltpu.VMEM((2, CW), jnp.int32),
            pltpu.VMEM((2, CW, H), f32),
            pltpu.VMEM((64, H), f32),
            pltpu.VMEM_SHARED((accr, H), f32),
            pltpu.SemaphoreType.DMA,
            pltpu.SemaphoreType.DMA,
            pltpu.SemaphoreType.DMA,
            pltpu.SemaphoreType.DMA,
        ],
    )

    BR = 1000
    grid_n = N // BR

    h1 = pl.pallas_call(
        _mm_tc,
        grid=(grid_n,),
        in_specs=[
            pl.BlockSpec((BR, D), lambda i: (i, 0)),
            pl.BlockSpec((D, H), lambda i: (0, 0)),
        ],
        out_specs=pl.BlockSpec((BR, H), lambda i: (i, 0)),
        out_shape=jax.ShapeDtypeStruct((N, H), f32),
    )(x, W1)

    degT = jnp.transpose(degp.reshape(NC, accr))
    dis_b, hs1 = pl.pallas_call(
        _dis_scale_tc,
        grid=(grid_n,),
        in_specs=[
            pl.BlockSpec((BR, NC), lambda i: (i, 0)),
            pl.BlockSpec((BR, H), lambda i: (i, 0)),
        ],
        out_specs=[
            pl.BlockSpec((BR, H), lambda i: (i, 0)),
            pl.BlockSpec((BR, H), lambda i: (i, 0)),
        ],
        out_shape=[
            jax.ShapeDtypeStruct((N, H), f32),
            jax.ShapeDtypeStruct((N, H), f32),
        ],
    )(degT, h1)

    acc1 = agg_call(hs1, edges1d, src_t, dst_t, zeros2)

    hs2 = pl.pallas_call(
        _mid_tc,
        grid=(grid_n,),
        in_specs=[
            pl.BlockSpec((1, BR, H), lambda i: (0, i, 0)),
            pl.BlockSpec((1, BR, H), lambda i: (1, i, 0)),
            pl.BlockSpec((BR, H), lambda i: (i, 0)),
            pl.BlockSpec((BR, H), lambda i: (i, 0)),
            pl.BlockSpec((1, H), lambda i: (0, 0)),
            pl.BlockSpec((H, H), lambda i: (0, 0)),
        ],
        out_specs=pl.BlockSpec((BR, H), lambda i: (i, 0)),
        out_shape=jax.ShapeDtypeStruct((N, H), f32),
    )(acc1, acc1, hs1, dis_b, b1.reshape(1, H), W2)

    acc2 = agg_call(hs2, edges1d, src_t, dst_t, zeros2)

    out = pl.pallas_call(
        _final_tc,
        grid=(grid_n,),
        in_specs=[
            pl.BlockSpec((1, BR, H), lambda i: (0, i, 0)),
            pl.BlockSpec((1, BR, H), lambda i: (1, i, 0)),
            pl.BlockSpec((BR, H), lambda i: (i, 0)),
            pl.BlockSpec((BR, H), lambda i: (i, 0)),
            pl.BlockSpec((1, H), lambda i: (0, 0)),
            pl.BlockSpec((1, 1, BR), lambda i: (i, 0, 0)),
            pl.BlockSpec((H, C), lambda i: (0, 0)),
            pl.BlockSpec((1, C), lambda i: (0, 0)),
        ],
        out_specs=pl.BlockSpec((G, C), lambda i: (0, 0)),
        out_shape=jax.ShapeDtypeStruct((G, C), f32),
        scratch_shapes=[
            pltpu.VMEM((G, H), f32),
            pltpu.VMEM((G, H), f32),
        ],
    )(acc2, acc2, hs2, dis_b, b2.reshape(1, H), batch.reshape(grid_n, 1, BR),
      Wc, bc.reshape(1, C))
    return out
